# baseline (speedup 1.0000x reference)
import jax
import jax.numpy as jnp
from jax.experimental import pallas as pl
from jax.experimental.pallas import tpu as pltpu

D_MODEL = 4096
NUM_PATHS = 512
K = 51
T_BLK = 256
V = 64


def _cex(ks, ix, i, j, flip):
    a, b = ks[i], ks[j]
    ia, ib = ix[i], ix[j]
    g = a < b
    if not flip:
        ks[i], ks[j] = jnp.where(g, b, a), jnp.where(g, a, b)
        ix[i], ix[j] = jnp.where(g, ib, ia), jnp.where(g, ia, ib)
    else:
        ks[i], ks[j] = jnp.where(g, a, b), jnp.where(g, b, a)
        ix[i], ix[j] = jnp.where(g, ia, ib), jnp.where(g, ib, ia)


def _sort64_desc(ks, ix):
    m = 2
    while m <= V:
        s = m // 2
        while s >= 1:
            for i in range(V):
                if i & s:
                    continue
                _cex(ks, ix, i, i | s, flip=bool(i & m) and m < V)
            s //= 2
        m *= 2


def _merge64_desc(ks, ix):
    s = V // 2
    while s >= 1:
        for i in range(V):
            if not i & s:
                _cex(ks, ix, i, i | s, flip=False)
        s //= 2


def _subrot(arr, d):
    return pltpu.roll(arr, 8 - d, axis=0)


def _gate_kernel(x_ref, w_ref, idx_ref, scores_ref, lg_ref):
    g = pl.program_id(0)
    n_steps = pl.num_programs(0)

    @pl.when(g < n_steps - 1)
    def _mm():
        lg_ref[g % 2] = jax.lax.dot_general(
            w_ref[...], x_ref[...], (((1,), (1,)), ((), ())),
            preferred_element_type=jnp.float32,
        )

    @pl.when(g > 0)
    def _sort():
        _topk_softmax(lg_ref[(g - 1) % 2], idx_ref, scores_ref)


def _topk_softmax(logits, idx_ref, scores_ref):
    t = logits.shape[-1]
    ks = [logits[8 * v: 8 * v + 8, :] for v in range(V)]
    sub = jax.lax.broadcasted_iota(jnp.int32, (8, t), 0)
    ix = [sub + 8 * v for v in range(V)]

    _sort64_desc(ks, ix)

    for d in (1, 2, 4):
        pks = [_subrot(ks[V - 1 - v], d) for v in range(V)]
        pix = [_subrot(ix[V - 1 - v], d) for v in range(V)]
        for v in range(V):
            g = pks[v] > ks[v]
            ks[v] = jnp.where(g, pks[v], ks[v])
            ix[v] = jnp.where(g, pix[v], ix[v])
        _merge64_desc(ks, ix)

    kv = jnp.concatenate([ks[r][0:1, :] for r in range(V)], axis=0)
    iv = jnp.concatenate([ix[r][0:1, :] for r in range(V)], axis=0)

    rank = jax.lax.broadcasted_iota(jnp.int32, (V, t), 0)
    e = jnp.where(rank < K, jnp.exp(kv - kv[0:1, :]), 0.0)
    ssum = jnp.sum(e, axis=0, keepdims=True)
    sc = e / ssum

    idx_ref[...] = iv
    scores_ref[...] = sc


@jax.jit
def kernel(x, W):
    n_tokens = x.shape[0]
    nblk = n_tokens // T_BLK
    grid = (nblk + 1,)
    idx_t, scores_t = pl.pallas_call(
        _gate_kernel,
        grid=grid,
        in_specs=[
            pl.BlockSpec(
                (T_BLK, D_MODEL), lambda i: (jnp.minimum(i, nblk - 1), 0)
            ),
            pl.BlockSpec((NUM_PATHS, D_MODEL), lambda i: (0, 0)),
        ],
        out_specs=[
            pl.BlockSpec((V, T_BLK), lambda i: (0, jnp.maximum(i - 1, 0))),
            pl.BlockSpec((V, T_BLK), lambda i: (0, jnp.maximum(i - 1, 0))),
        ],
        out_shape=[
            jax.ShapeDtypeStruct((V, n_tokens), jnp.int32),
            jax.ShapeDtypeStruct((V, n_tokens), jnp.float32),
        ],
        scratch_shapes=[
            pltpu.VMEM((2, NUM_PATHS, T_BLK), jnp.float32),
        ],
    )(x, W)
    return idx_t[:K].T, scores_t[:K].T


# speedup vs baseline: 1.1262x; 1.1262x over previous
import jax
import jax.numpy as jnp
from jax.experimental import pallas as pl
from jax.experimental.pallas import tpu as pltpu

D_MODEL = 4096
NUM_PATHS = 512
K = 51
T_BLK = 256
V = 64


def _cex(ks, ix, i, j, flip):
    a, b = ks[i], ks[j]
    ia, ib = ix[i], ix[j]
    g = a < b
    if not flip:
        ks[i], ks[j] = jnp.where(g, b, a), jnp.where(g, a, b)
        ix[i], ix[j] = jnp.where(g, ib, ia), jnp.where(g, ia, ib)
    else:
        ks[i], ks[j] = jnp.where(g, a, b), jnp.where(g, b, a)
        ix[i], ix[j] = jnp.where(g, ia, ib), jnp.where(g, ib, ia)


def _sort64_desc(ks, ix):
    def merge(lo, hi, r):
        step = r * 2
        if step < hi - lo:
            merge(lo, hi, step)
            merge(lo + r, hi, step)
            for i in range(lo + r, hi - r, step):
                _cex(ks, ix, i, i + r, flip=False)
        else:
            _cex(ks, ix, lo, lo + r, flip=False)

    def msort(lo, hi):
        if hi - lo > 1:
            mid = (lo + hi) // 2
            msort(lo, mid)
            msort(mid, hi)
            merge(lo, hi, 1)

    msort(0, V)


def _merge64_desc(ks, ix):
    s = V // 2
    while s >= 1:
        for i in range(V):
            if not i & s:
                _cex(ks, ix, i, i | s, flip=False)
        s //= 2


def _subrot(arr, d):
    return pltpu.roll(arr, 8 - d, axis=0)


def _gate_kernel(x_ref, w_ref, idx_ref, scores_ref):
    logits = jax.lax.dot_general(
        w_ref[...], x_ref[...], (((1,), (1,)), ((), ())),
        preferred_element_type=jnp.float32,
    )
    t = logits.shape[-1]
    ks = [logits[8 * v: 8 * v + 8, :] for v in range(V)]
    sub = jax.lax.broadcasted_iota(jnp.int32, (8, t), 0)
    ix = [sub + 8 * v for v in range(V)]

    _sort64_desc(ks, ix)

    for d in (1, 2, 4):
        pks = [_subrot(ks[V - 1 - v], d) for v in range(V)]
        pix = [_subrot(ix[V - 1 - v], d) for v in range(V)]
        for v in range(V):
            g = pks[v] > ks[v]
            ks[v] = jnp.where(g, pks[v], ks[v])
            ix[v] = jnp.where(g, pix[v], ix[v])
        _merge64_desc(ks, ix)

    kv = jnp.concatenate([ks[r][0:1, :] for r in range(V)], axis=0)
    iv = jnp.concatenate([ix[r][0:1, :] for r in range(V)], axis=0)

    rank = jax.lax.broadcasted_iota(jnp.int32, (V, t), 0)
    e = jnp.where(rank < K, jnp.exp(kv - kv[0:1, :]), 0.0)
    ssum = jnp.sum(e, axis=0, keepdims=True)
    sc = e / ssum

    idx_ref[...] = iv
    scores_ref[...] = sc


@jax.jit
def kernel(x, W):
    n_tokens = x.shape[0]
    nblk = n_tokens // T_BLK
    idx_t, scores_t = pl.pallas_call(
        _gate_kernel,
        grid=(nblk,),
        in_specs=[
            pl.BlockSpec((T_BLK, D_MODEL), lambda i: (i, 0)),
            pl.BlockSpec((NUM_PATHS, D_MODEL), lambda i: (0, 0)),
        ],
        out_specs=[
            pl.BlockSpec((V, T_BLK), lambda i: (0, i)),
            pl.BlockSpec((V, T_BLK), lambda i: (0, i)),
        ],
        out_shape=[
            jax.ShapeDtypeStruct((V, n_tokens), jnp.int32),
            jax.ShapeDtypeStruct((V, n_tokens), jnp.float32),
        ],
    )(x, W)
    return idx_t[:K].T, scores_t[:K].T
